# Initial kernel scaffold; baseline (speedup 1.0000x reference)
import jax
import jax.numpy as jnp
from jax.experimental import pallas as pl


def kernel(input, We0, be0, We1, be1, We2, be2, Wd0, bd0, Wd1, bd1, Wd2, bd2, codebook):
    raise NotImplementedError("write your pallas kernel here")


# speedup vs baseline: 1.1719x; 1.1719x over previous
import jax
import jax.numpy as jnp
from jax.experimental import pallas as pl
from jax.experimental.pallas import tpu as pltpu

B = 16384
COND_DIM = 256
ACT_DIM = 32
DIN = COND_DIM + ACT_DIM
H0, H1 = 512, 256
EMB = 64
K = 1024
COMMIT = 0.25

BLK = 1024
NB = B // BLK


def _elu(x):
    return jnp.where(x > 0, x, jnp.exp(x) - 1.0)


def _body(x_ref, We0_ref, be0_ref, We1_ref, be1_ref, We2_ref, be2_ref,
          Wd0_ref, bd0_ref, Wd1_ref, bd1_ref, Wd2_ref, bd2_ref, cb_ref,
          recon_ref, idx_ref, stats_ref,
          counts_ref, acc_ref):
    i = pl.program_id(0)

    @pl.when(i == 0)
    def _init():
        counts_ref[...] = jnp.zeros_like(counts_ref)
        acc_ref[0] = 0.0
        acc_ref[1] = 0.0

    x = x_ref[...]
    cond = x[:, :COND_DIM]
    act = x[:, COND_DIM:]

    h = _elu(jnp.dot(x, We0_ref[...], preferred_element_type=jnp.float32)
             + be0_ref[...])
    h = _elu(jnp.dot(h, We1_ref[...], preferred_element_type=jnp.float32)
             + be1_ref[...])
    z = (jnp.dot(h, We2_ref[...], preferred_element_type=jnp.float32)
         + be2_ref[...])

    zn = z / (jnp.sqrt(jnp.sum(z * z, axis=-1, keepdims=True)) + 1e-12)
    cb = cb_ref[...]
    cbn = cb / (jnp.sqrt(jnp.sum(cb * cb, axis=-1, keepdims=True)) + 1e-12)
    dist = jax.lax.dot_general(zn, cbn, (((1,), (1,)), ((), ())),
                               preferred_element_type=jnp.float32)

    idx = jnp.argmax(dist, axis=-1).astype(jnp.int32)
    idx_ref[...] = idx[:, None]

    onehot = (jax.lax.broadcasted_iota(jnp.int32, (BLK, K), 1)
              == idx[:, None]).astype(jnp.float32)
    counts_ref[...] += jnp.sum(onehot, axis=0, keepdims=True)

    quantized = jnp.dot(onehot, cb, preferred_element_type=jnp.float32)
    qd = quantized - z
    acc_ref[0] += jnp.sum(qd * qd)

    d = _elu(jnp.dot(cond, Wd0_ref[:COND_DIM, :],
                     preferred_element_type=jnp.float32)
             + jnp.dot(quantized, Wd0_ref[COND_DIM:, :],
                       preferred_element_type=jnp.float32)
             + bd0_ref[...])
    d = _elu(jnp.dot(d, Wd1_ref[...], preferred_element_type=jnp.float32)
             + bd1_ref[...])
    recon = (jnp.dot(d, Wd2_ref[...], preferred_element_type=jnp.float32)
             + bd2_ref[...])
    recon_ref[...] = recon

    rd = recon - act
    acc_ref[1] += jnp.sum(rd * rd)

    @pl.when(i == NB - 1)
    def _finalize():
        q_loss = acc_ref[0] / (B * EMB)
        rec_loss = acc_ref[1] / (B * ACT_DIM)
        p = counts_ref[...] / B
        perp = jnp.exp(-jnp.sum(p * jnp.log(p + 1e-10)))
        lane = jax.lax.broadcasted_iota(jnp.int32, (1, 128), 1)
        out = jnp.where(lane == 0, q_loss,
              jnp.where(lane == 1, COMMIT * q_loss,
              jnp.where(lane == 2, rec_loss, perp)))
        stats_ref[...] = out


def kernel(input, We0, be0, We1, be1, We2, be2,
           Wd0, bd0, Wd1, bd1, Wd2, bd2, codebook):
    full = lambda shape: pl.BlockSpec(shape, lambda i: (0,) * len(shape))
    recon, idx2d, stats = pl.pallas_call(
        _body,
        grid=(NB,),
        in_specs=[
            pl.BlockSpec((BLK, DIN), lambda i: (i, 0)),
            full((DIN, H0)), full((H0,)),
            full((H0, H1)), full((H1,)),
            full((H1, EMB)), full((EMB,)),
            full((COND_DIM + EMB, H1)), full((H1,)),
            full((H1, H0)), full((H0,)),
            full((H0, ACT_DIM)), full((ACT_DIM,)),
            full((K, EMB)),
        ],
        out_specs=[
            pl.BlockSpec((BLK, ACT_DIM), lambda i: (i, 0)),
            pl.BlockSpec((BLK, 1), lambda i: (i, 0)),
            pl.BlockSpec((1, 128), lambda i: (0, 0)),
        ],
        out_shape=[
            jax.ShapeDtypeStruct((B, ACT_DIM), jnp.float32),
            jax.ShapeDtypeStruct((B, 1), jnp.int32),
            jax.ShapeDtypeStruct((1, 128), jnp.float32),
        ],
        scratch_shapes=[
            pltpu.VMEM((1, K), jnp.float32),
            pltpu.SMEM((2,), jnp.float32),
        ],
    )(input, We0, be0, We1, be1, We2, be2,
      Wd0, bd0, Wd1, bd1, Wd2, bd2, codebook)
    q_loss = stats[0, 0]
    e_loss = stats[0, 1]
    rec_loss = stats[0, 2]
    perp = stats[0, 3]
    return (recon, idx2d[:, 0], q_loss, e_loss, rec_loss, perp)


# speedup vs baseline: 1.1868x; 1.0127x over previous
import jax
import jax.numpy as jnp
from jax.experimental import pallas as pl
from jax.experimental.pallas import tpu as pltpu

B = 16384
COND_DIM = 256
ACT_DIM = 32
DIN = COND_DIM + ACT_DIM
H0, H1 = 512, 256
EMB = 64
K = 1024
COMMIT = 0.25

BLK = 2048
NB = B // BLK
SUB = BLK // 8


def _elu(x):
    return jnp.where(x > 0, x, jnp.exp(x) - 1.0)


def _bdot(a, b):
    return jnp.dot(a.astype(jnp.bfloat16), b.astype(jnp.bfloat16),
                   preferred_element_type=jnp.float32)


def _body(x_ref, We0_ref, be0_ref, We1_ref, be1_ref, We2_ref, be2_ref,
          Wd0_ref, bd0_ref, Wd1_ref, bd1_ref, Wd2_ref, bd2_ref, cb_ref,
          recon_ref, idx_ref, stats_ref,
          counts_ref, qacc_ref, racc_ref):
    i = pl.program_id(0)

    @pl.when(i == 0)
    def _init():
        counts_ref[...] = jnp.zeros_like(counts_ref)
        qacc_ref[...] = jnp.zeros_like(qacc_ref)
        racc_ref[...] = jnp.zeros_like(racc_ref)

    x = x_ref[...]
    cond = x[:, :COND_DIM]
    act = x[:, COND_DIM:]

    h = _elu(jnp.dot(x, We0_ref[...], preferred_element_type=jnp.float32)
             + be0_ref[...])
    h = _elu(jnp.dot(h, We1_ref[...], preferred_element_type=jnp.float32)
             + be1_ref[...])
    z = (jnp.dot(h, We2_ref[...], preferred_element_type=jnp.float32)
         + be2_ref[...])

    zn = z / (jnp.sqrt(jnp.sum(z * z, axis=-1, keepdims=True)) + 1e-12)
    cb = cb_ref[...]
    cbn = cb / (jnp.sqrt(jnp.sum(cb * cb, axis=-1, keepdims=True)) + 1e-12)
    dist = jax.lax.dot_general(zn, cbn, (((1,), (1,)), ((), ())),
                               preferred_element_type=jnp.float32)

    idx = jnp.argmax(dist, axis=-1).astype(jnp.int32)
    idx_ref[...] = idx[:, None]

    onehot = (jax.lax.broadcasted_iota(jnp.int32, (BLK, K), 1)
              == idx[:, None]).astype(jnp.float32)
    counts_ref[...] += jnp.sum(onehot.reshape(SUB, 8, K), axis=0)

    quantized = jnp.dot(onehot, cb, preferred_element_type=jnp.float32)
    qd = quantized - z
    qacc_ref[...] += jnp.sum((qd * qd).reshape(SUB, 8, EMB), axis=0)

    d = _elu(_bdot(cond, Wd0_ref[:COND_DIM, :])
             + _bdot(quantized, Wd0_ref[COND_DIM:, :])
             + bd0_ref[...])
    d = _elu(_bdot(d, Wd1_ref[...]) + bd1_ref[...])
    recon = _bdot(d, Wd2_ref[...]) + bd2_ref[...]
    recon_ref[...] = recon

    rd = recon - act
    racc_ref[...] += jnp.sum((rd * rd).reshape(SUB, 8, ACT_DIM), axis=0)

    @pl.when(i == NB - 1)
    def _finalize():
        q_loss = jnp.sum(qacc_ref[...]) / (B * EMB)
        rec_loss = jnp.sum(racc_ref[...]) / (B * ACT_DIM)
        p = jnp.sum(counts_ref[...], axis=0, keepdims=True) / B
        perp = jnp.exp(-jnp.sum(p * jnp.log(p + 1e-10)))
        lane = jax.lax.broadcasted_iota(jnp.int32, (1, 128), 1)
        out = jnp.where(lane == 0, q_loss,
              jnp.where(lane == 1, COMMIT * q_loss,
              jnp.where(lane == 2, rec_loss, perp)))
        stats_ref[...] = out


def kernel(input, We0, be0, We1, be1, We2, be2,
           Wd0, bd0, Wd1, bd1, Wd2, bd2, codebook):
    full = lambda shape: pl.BlockSpec(shape, lambda i: (0,) * len(shape))
    recon, idx2d, stats = pl.pallas_call(
        _body,
        grid=(NB,),
        in_specs=[
            pl.BlockSpec((BLK, DIN), lambda i: (i, 0)),
            full((DIN, H0)), full((H0,)),
            full((H0, H1)), full((H1,)),
            full((H1, EMB)), full((EMB,)),
            full((COND_DIM + EMB, H1)), full((H1,)),
            full((H1, H0)), full((H0,)),
            full((H0, ACT_DIM)), full((ACT_DIM,)),
            full((K, EMB)),
        ],
        out_specs=[
            pl.BlockSpec((BLK, ACT_DIM), lambda i: (i, 0)),
            pl.BlockSpec((BLK, 1), lambda i: (i, 0)),
            pl.BlockSpec((1, 128), lambda i: (0, 0)),
        ],
        out_shape=[
            jax.ShapeDtypeStruct((B, ACT_DIM), jnp.float32),
            jax.ShapeDtypeStruct((B, 1), jnp.int32),
            jax.ShapeDtypeStruct((1, 128), jnp.float32),
        ],
        scratch_shapes=[
            pltpu.VMEM((8, K), jnp.float32),
            pltpu.VMEM((8, EMB), jnp.float32),
            pltpu.VMEM((8, ACT_DIM), jnp.float32),
        ],
    )(input, We0, be0, We1, be1, We2, be2,
      Wd0, bd0, Wd1, bd1, Wd2, bd2, codebook)
    q_loss = stats[0, 0]
    e_loss = stats[0, 1]
    rec_loss = stats[0, 2]
    perp = stats[0, 3]
    return (recon, idx2d[:, 0], q_loss, e_loss, rec_loss, perp)
